# baseline (speedup 1.0000x reference)
import functools

import jax
import jax.numpy as jnp
from jax import lax
from jax.experimental import pallas as pl
from jax.experimental.pallas import tpu as pltpu
from jax.experimental.pallas import tpu_sc as plsc

BATCH = 16384
EMBED_DIM = 128
NUM_WORKERS = 32
PAIRS_PER_WORKER = BATCH // NUM_WORKERS
CHUNK = 128
NUM_CHUNKS = PAIRS_PER_WORKER // CHUNK
IDX_ROWS_PER_WORKER = PAIRS_PER_WORKER // CHUNK


def _sc_kernel(users_hbm, items_hbm, user_table, item_table, out_hbm,
               idx_u, idx_i, rows_u, rows_i, out_v, sem_u, sem_i):
    wid = lax.axis_index("s") * 2 + lax.axis_index("c")
    row0 = wid * IDX_ROWS_PER_WORKER

    pltpu.sync_copy(users_hbm.at[pl.ds(row0, IDX_ROWS_PER_WORKER)], idx_u)
    pltpu.sync_copy(items_hbm.at[pl.ds(row0, IDX_ROWS_PER_WORKER)], idx_i)

    lane = lax.iota(jnp.int32, 16)
    rots = [jnp.bitwise_and(lane + (1 << s), 15) for s in range(4)]
    groups_per_chunk = CHUNK // 16

    def lane_sum(v):
        for r in rots:
            v = v + jnp.take_along_axis(v, r, axis=0)
        return v

    for c in range(NUM_CHUNKS):
        cu = pltpu.async_copy(user_table.at[idx_u.at[c]], rows_u, sem_u)
        ci = pltpu.async_copy(item_table.at[idx_i.at[c]], rows_i, sem_i)
        cu.wait()
        ci.wait()

        def group_body(g, carry, c=c):
            res = jnp.zeros((16,), jnp.float32)
            for j in range(16):
                p = g * 16 + j
                acc = rows_u[p, pl.ds(0, 16)] * rows_i[p, pl.ds(0, 16)]
                for k in range(1, EMBED_DIM // 16):
                    acc = acc + rows_u[p, pl.ds(k * 16, 16)] * rows_i[p, pl.ds(k * 16, 16)]
                res = jnp.where(lane == j, lane_sum(acc), res)
            out_v[c * groups_per_chunk + g] = 1.0 / (1.0 + jnp.exp(-res))
            return carry

        lax.fori_loop(0, groups_per_chunk, group_body, 0)

    pltpu.sync_copy(
        out_v, out_hbm.at[pl.ds(wid * (PAIRS_PER_WORKER // 16), PAIRS_PER_WORKER // 16)])


@functools.partial(jax.jit, static_argnums=())
def _run(users2d, items2d, user_table, item_table):
    mesh = plsc.VectorSubcoreMesh(core_axis_name="c", subcore_axis_name="s")
    f = pl.kernel(
        _sc_kernel,
        mesh=mesh,
        out_type=jax.ShapeDtypeStruct((BATCH // 16, 16), jnp.float32),
        scratch_types=[
            pltpu.VMEM((IDX_ROWS_PER_WORKER, CHUNK), jnp.int32),
            pltpu.VMEM((IDX_ROWS_PER_WORKER, CHUNK), jnp.int32),
            pltpu.VMEM((CHUNK, EMBED_DIM), jnp.float32),
            pltpu.VMEM((CHUNK, EMBED_DIM), jnp.float32),
            pltpu.VMEM((PAIRS_PER_WORKER // 16, 16), jnp.float32),
            pltpu.SemaphoreType.DMA,
            pltpu.SemaphoreType.DMA,
        ],
    )
    return f(users2d, items2d, user_table, item_table)


def kernel(data, user_table, item_table):
    users2d = data[:, 0].reshape(BATCH // CHUNK, CHUNK)
    items2d = data[:, 1].reshape(BATCH // CHUNK, CHUNK)
    return _run(users2d, items2d, user_table, item_table).reshape(BATCH)
